# Initial kernel scaffold; baseline (speedup 1.0000x reference)
import functools

import jax
import jax.numpy as jnp
from jax.experimental import pallas as pl
from jax.experimental.pallas import tpu as pltpu

H = 8
KNN = 9
RB = 256

_NEG = -3.0e38


def _attn_kernel(h_blk_ref, h_all_ref, wq_ref, bq_ref, wk_ref, bk_ref,
                 wv_ref, bv_ref, out_ref, kv_ref, sqa_ref, *, n, c, hd):
    rb = pl.program_id(1)
    d = hd // H
    scale = float(d) ** 0.5

    @pl.when(rb == 0)
    def _init_batch():
        h_all = h_all_ref[0]
        kh = jax.lax.dot_general(
            h_all, wk_ref[...], (((1,), (1,)), ((), ())),
            preferred_element_type=jnp.float32,
            precision=jax.lax.Precision.HIGHEST) + bk_ref[...]
        vh = jax.lax.dot_general(
            h_all, wv_ref[...], (((1,), (1,)), ((), ())),
            preferred_element_type=jnp.float32,
            precision=jax.lax.Precision.HIGHEST) + bv_ref[...]
        kv_ref[0] = kh
        kv_ref[1] = vh
        ones = jnp.ones((1, c), dtype=jnp.float32)
        sqa_ref[...] = jax.lax.dot_general(
            ones, h_all * h_all, (((1,), (1,)), ((), ())),
            preferred_element_type=jnp.float32,
            precision=jax.lax.Precision.HIGHEST)

    h_blk = h_blk_ref[0]
    qh = jax.lax.dot_general(
        h_blk, wq_ref[...], (((1,), (1,)), ((), ())),
        preferred_element_type=jnp.float32,
        precision=jax.lax.Precision.HIGHEST) + bq_ref[...]

    inner = jax.lax.dot_general(
        h_blk, h_all_ref[0], (((1,), (1,)), ((), ())),
        preferred_element_type=jnp.float32,
        precision=jax.lax.Precision.HIGHEST)
    adj = 2.0 * inner - sqa_ref[...]

    x = adj
    thr = None
    for _ in range(KNN):
        thr = jnp.max(x, axis=1, keepdims=True)
        x = jnp.where(x >= thr, _NEG, x)
    mask = adj >= thr

    kh = kv_ref[0]
    vh = kv_ref[1]
    for head in range(H):
        sl = slice(head * d, (head + 1) * d)
        s = jax.lax.dot_general(
            qh[:, sl], kh[:, sl], (((1,), (1,)), ((), ())),
            preferred_element_type=jnp.float32,
            precision=jax.lax.Precision.HIGHEST)
        s = jnp.exp(jnp.clip(s / scale, -5.0, 5.0))
        s = jnp.where(mask, s, 0.0)
        wv = jax.lax.dot_general(
            s, vh[:, sl], (((1,), (0,)), ((), ())),
            preferred_element_type=jnp.float32,
            precision=jax.lax.Precision.HIGHEST)
        z = jnp.sum(s, axis=1, keepdims=True)
        out_ref[0, :, sl] = wv / z


def kernel(h, WQ, bQ, WK, bK, WV, bV):
    b, n, c = h.shape
    hd = WQ.shape[0]
    nb = n // RB

    grid = (b, nb)
    out = pl.pallas_call(
        functools.partial(_attn_kernel, n=n, c=c, hd=hd),
        grid=grid,
        in_specs=[
            pl.BlockSpec((1, RB, c), lambda bi, ri: (bi, ri, 0)),
            pl.BlockSpec((1, n, c), lambda bi, ri: (bi, 0, 0)),
            pl.BlockSpec((hd, c), lambda bi, ri: (0, 0)),
            pl.BlockSpec((1, hd), lambda bi, ri: (0, 0)),
            pl.BlockSpec((hd, c), lambda bi, ri: (0, 0)),
            pl.BlockSpec((1, hd), lambda bi, ri: (0, 0)),
            pl.BlockSpec((hd, c), lambda bi, ri: (0, 0)),
            pl.BlockSpec((1, hd), lambda bi, ri: (0, 0)),
        ],
        out_specs=pl.BlockSpec((1, RB, hd), lambda bi, ri: (bi, ri, 0)),
        out_shape=jax.ShapeDtypeStruct((b, n, hd), jnp.float32),
        scratch_shapes=[
            pltpu.VMEM((2, n, hd), jnp.float32),
            pltpu.VMEM((1, n), jnp.float32),
        ],
    )(h, h, WQ, bQ.reshape(1, hd), WK, bK.reshape(1, hd), WV,
      bV.reshape(1, hd))
    return out


# speedup vs baseline: 13.5270x; 13.5270x over previous
import functools

import jax
import jax.numpy as jnp
from jax.experimental import pallas as pl
from jax.experimental.pallas import tpu as pltpu

H = 8
KNN = 9
RB = 128
CK = 512

_NEG = -3.0e38
_NT = (((1,), (1,)), ((), ()))
_NN = (((1,), (0,)), ((), ()))


def _bdot(a, b, dims):
    return jax.lax.dot_general(a, b, dims,
                               preferred_element_type=jnp.float32)


def _split2(x):
    hi = x.astype(jnp.bfloat16)
    lo = (x - hi.astype(jnp.float32)).astype(jnp.bfloat16)
    return hi, lo


def _proj_head(h0, w_ref, b_ref, head, d):
    w_hi = w_ref[head * d:(head + 1) * d, :].astype(jnp.bfloat16)
    return _bdot(h0, w_hi, _NT) + b_ref[head]


def _attn_kernel(h_blk_ref, h_all_ref, sqr_ref, sqc_ref, wq_ref, bq_ref,
                 wk_ref, bk_ref, wv_ref, bv_ref, out_ref, hs_ref, ks_ref,
                 vs_ref, adj_ref, x_ref, *, n, c, hd):
    rb = pl.program_id(1)
    d = hd // H
    scale = float(d) ** 0.5

    @pl.when(rb == 0)
    def _init_batch():
        for j in range(0, n, CK):
            hc = h_all_ref[0, j:j + CK, :]
            h0 = hc.astype(jnp.bfloat16)
            hs_ref[0, j:j + CK] = h0
            for head in range(H):
                kh = _proj_head(h0, wk_ref, bk_ref, head, d)
                k0, k1 = _split2(kh)
                ks_ref[0, head, j:j + CK] = k0
                ks_ref[1, head, j:j + CK] = k1
                vh = _proj_head(h0, wv_ref, bv_ref, head, d)
                v0, v1 = _split2(vh)
                vs_ref[0, head, j:j + CK] = v0
                vs_ref[1, head, j:j + CK] = v1

    b0 = h_blk_ref[0].astype(jnp.bfloat16)

    inner = _bdot(b0, hs_ref[0], _NT)
    adj_ref[...] = (2.0 * inner - sqr_ref[0]) - sqc_ref[0]

    x_ref[...] = adj_ref[...]
    thr = None
    for _ in range(KNN):
        thr = jnp.max(x_ref[...], axis=1, keepdims=True)
        x_ref[...] = jnp.where(x_ref[...] >= thr, _NEG, x_ref[...])

    for head in range(H):
        qh = _proj_head(b0, wq_ref, bq_ref, head, d)
        q0, q1 = _split2(qh)
        s = (_bdot(q0, ks_ref[1, head], _NT)
             + _bdot(q1, ks_ref[0, head], _NT)) \
            + _bdot(q0, ks_ref[0, head], _NT)
        s = jnp.exp(jnp.clip(s / scale, -5.0, 5.0))
        s = jnp.where(adj_ref[...] >= thr, s, 0.0)
        s0, s1 = _split2(s)
        wv = (_bdot(s0, vs_ref[1, head], _NN)
              + _bdot(s1, vs_ref[0, head], _NN)) \
            + _bdot(s0, vs_ref[0, head], _NN)
        z = jnp.sum(s, axis=1, keepdims=True)
        out_ref[0, head] = wv / z


def kernel(h, WQ, bQ, WK, bK, WV, bV):
    b, n, c = h.shape
    hd = WQ.shape[0]
    d = hd // H
    nb = n // RB

    sq = jnp.sum(h * h, axis=-1)

    grid = (b, nb)
    out = pl.pallas_call(
        functools.partial(_attn_kernel, n=n, c=c, hd=hd),
        grid=grid,
        in_specs=[
            pl.BlockSpec((1, RB, c), lambda bi, ri: (bi, ri, 0)),
            pl.BlockSpec((1, n, c), lambda bi, ri: (bi, 0, 0)),
            pl.BlockSpec((1, RB, 1), lambda bi, ri: (bi, ri, 0)),
            pl.BlockSpec((1, 1, n), lambda bi, ri: (bi, 0, 0)),
            pl.BlockSpec((hd, c), lambda bi, ri: (0, 0)),
            pl.BlockSpec((H, 1, d), lambda bi, ri: (0, 0, 0)),
            pl.BlockSpec((hd, c), lambda bi, ri: (0, 0)),
            pl.BlockSpec((H, 1, d), lambda bi, ri: (0, 0, 0)),
            pl.BlockSpec((hd, c), lambda bi, ri: (0, 0)),
            pl.BlockSpec((H, 1, d), lambda bi, ri: (0, 0, 0)),
        ],
        out_specs=pl.BlockSpec((1, H, RB, d), lambda bi, ri: (bi, 0, ri, 0)),
        out_shape=jax.ShapeDtypeStruct((b, H, n, d), jnp.float32),
        scratch_shapes=[
            pltpu.VMEM((1, n, c), jnp.bfloat16),
            pltpu.VMEM((2, H, n, d), jnp.bfloat16),
            pltpu.VMEM((2, H, n, d), jnp.bfloat16),
            pltpu.VMEM((RB, n), jnp.float32),
            pltpu.VMEM((RB, n), jnp.float32),
        ],
    )(h, h, sq.reshape(b, n, 1), sq.reshape(b, 1, n), WQ, bQ.reshape(H, 1, d), WK,
      bK.reshape(H, 1, d), WV, bV.reshape(H, 1, d))
    return out.transpose(0, 2, 1, 3).reshape(b, n, hd)
